# Initial kernel scaffold; baseline (speedup 1.0000x reference)
import jax
import jax.numpy as jnp
from jax.experimental import pallas as pl


def kernel(x, edge_index, curr_idx, dest_idx, neighbor_indices, Wl1, bl1, Wr1, Wl2, bl2, Wr2, W1, b1, W2, b2):
    raise NotImplementedError("write your pallas kernel here")


# speedup vs baseline: 4.8178x; 4.8178x over previous
import functools

import jax
import jax.numpy as jnp
from jax import lax
from jax.experimental import pallas as pl
from jax.experimental.pallas import tpu as pltpu
from jax.experimental.pallas import tpu_sc as plsc

N = 10000
E = 320000
F = 128
H = 128
K = 64

NC = 2
NS = 16
FH = F // NC
EPT = E // NS
CH = 80
NCH = EPT // CH
NPAD = 10240
ZR = NPAD // NS
CR = 624
CR_LAST = N - CR * (NS - 1)
NDEG = 10240

_PREC = jax.lax.Precision.HIGHEST


def _seg_body(ta, tb, srcR, dstR, zrows, acc_out, deg_out,
              src_buf, dst_buf, rows, deg_loc, acc, sem):
    core = lax.axis_index("c")
    sub = lax.axis_index("s")

    pltpu.sync_copy(zrows.at[pl.ds(sub * ZR, ZR)], acc.at[pl.ds(sub * ZR, ZR)])

    zero16 = jnp.zeros((16,), jnp.float32)

    @pl.when(core == 0)
    def _():
        def _zdeg(i, _):
            deg_loc[pl.ds(i * 16, 16)] = zero16
            return 0

        lax.fori_loop(0, NDEG // 16, _zdeg, 0)

    pltpu.sync_copy(srcR.at[sub], src_buf)
    pltpu.sync_copy(dstR.at[sub], dst_buf)

    plsc.subcore_barrier()

    ones16 = jnp.full((16,), 1.0, jnp.float32)

    def _chunk(j, _):
        @pl.when(core == 0)
        def _():
            pltpu.async_copy(ta.at[src_buf.at[j]], rows, sem).wait()

        @pl.when(core == 1)
        def _():
            pltpu.async_copy(tb.at[src_buf.at[j]], rows, sem).wait()

        pltpu.sync_copy(rows, acc.at[dst_buf.at[j]], add=True)

        @pl.when(core == 0)
        def _():
            for q in range(CH // 16):
                d = dst_buf[j, pl.ds(q * 16, 16)]
                plsc.addupdate_scatter(deg_loc, [d], ones16)

        return 0

    lax.fori_loop(0, NCH, _chunk, 0)

    plsc.subcore_barrier()

    @pl.when(sub < NS - 1)
    def _():
        pltpu.sync_copy(acc.at[pl.ds(sub * CR, CR)],
                        acc_out.at[core, pl.ds(sub * CR, CR)])

    @pl.when(sub == NS - 1)
    def _():
        pltpu.sync_copy(acc.at[pl.ds((NS - 1) * CR, CR_LAST)],
                        acc_out.at[core, pl.ds((NS - 1) * CR, CR_LAST)])

    @pl.when(core == 0)
    def _():
        pltpu.sync_copy(deg_loc, deg_out.at[pl.ds(sub * NDEG, NDEG)])


_seg = pl.kernel(
    _seg_body,
    out_type=(
        jax.ShapeDtypeStruct((NC, N, FH), jnp.float32),
        jax.ShapeDtypeStruct((NS * NDEG,), jnp.float32),
    ),
    mesh=plsc.VectorSubcoreMesh(core_axis_name="c", subcore_axis_name="s"),
    scratch_types=(
        pltpu.VMEM((NCH, CH), jnp.int32),
        pltpu.VMEM((NCH, CH), jnp.int32),
        pltpu.VMEM((CH, FH), jnp.float32),
        pltpu.VMEM((NDEG,), jnp.float32),
        pltpu.VMEM_SHARED((NPAD, FH), jnp.float32),
        pltpu.SemaphoreType.DMA,
    ),
    compiler_params=pltpu.CompilerParams(needs_layout_passes=False,
                                         use_tc_tiling_on_sc=False),
)


def _gt_body(embs, tidx, out, tidx_v, rows, sem):
    core = lax.axis_index("c")
    sub = lax.axis_index("s")

    @pl.when(jnp.logical_and(core == 0, sub == 0))
    def _():
        pltpu.sync_copy(tidx, tidx_v)
        pltpu.async_copy(embs.at[tidx_v], rows, sem).wait()
        pltpu.sync_copy(rows, out)


_gather_t = pl.kernel(
    _gt_body,
    out_type=jax.ShapeDtypeStruct((80, F), jnp.float32),
    mesh=plsc.VectorSubcoreMesh(core_axis_name="c", subcore_axis_name="s"),
    scratch_types=(
        pltpu.VMEM((80,), jnp.int32),
        pltpu.VMEM((80, F), jnp.float32),
        pltpu.SemaphoreType.DMA,
    ),
    compiler_params=pltpu.CompilerParams(needs_layout_passes=False),
)


def _dense_tile(aggp_ref, degp_ref, x_ref, wl_ref, wr_ref, b_ref, o_ref, *,
                act, split_in, split_out):
    deg = jnp.sum(degp_ref[...], axis=1)
    dinv = 1.0 / jnp.clip(deg, 1.0)[:, None]
    m0 = aggp_ref[0] * dinv
    m1 = aggp_ref[1] * dinv
    y = (lax.dot_general(m0, wl_ref[:, :FH], (((1,), (1,)), ((), ())),
                         precision=_PREC)
         + lax.dot_general(m1, wl_ref[:, FH:], (((1,), (1,)), ((), ())),
                           precision=_PREC)
         + b_ref[...])
    if split_in:
        y = (y
             + lax.dot_general(x_ref[0], wr_ref[:, :FH], (((1,), (1,)), ((), ())),
                               precision=_PREC)
             + lax.dot_general(x_ref[1], wr_ref[:, FH:], (((1,), (1,)), ((), ())),
                               precision=_PREC))
    else:
        y = y + lax.dot_general(x_ref[...], wr_ref[...], (((1,), (1,)), ((), ())),
                                precision=_PREC)
    if act:
        y = jnp.maximum(y, 0.0)
    if split_out:
        o_ref[0] = y[:, :FH]
        o_ref[1] = y[:, FH:]
    else:
        o_ref[...] = y


def _dense(aggp, degp_t, x, wl, b, wr, act, split_in, split_out):
    blk = 1000
    grid = N // blk
    x_spec = (pl.BlockSpec((NC, blk, FH), lambda i: (0, i, 0)) if split_in
              else pl.BlockSpec((blk, F), lambda i: (i, 0)))
    if split_out:
        out_spec = pl.BlockSpec((NC, blk, FH), lambda i: (0, i, 0))
        out_shape = jax.ShapeDtypeStruct((NC, N, FH), jnp.float32)
    else:
        out_spec = pl.BlockSpec((blk, H), lambda i: (i, 0))
        out_shape = jax.ShapeDtypeStruct((N, H), jnp.float32)
    return pl.pallas_call(
        functools.partial(_dense_tile, act=act, split_in=split_in,
                          split_out=split_out),
        grid=(grid,),
        in_specs=[
            pl.BlockSpec((NC, blk, FH), lambda i: (0, i, 0)),
            pl.BlockSpec((blk, NS), lambda i: (i, 0)),
            x_spec,
            pl.BlockSpec((H, F), lambda i: (0, 0)),
            pl.BlockSpec((H, F), lambda i: (0, 0)),
            pl.BlockSpec((1, H), lambda i: (0, 0)),
        ],
        out_specs=out_spec,
        out_shape=out_shape,
    )(aggp, degp_t, x, wl, wr, b.reshape(1, H))


def _final_tile(et_ref, w1_ref, b1_ref, w2_ref, b2_ref, o_ref):
    curr = et_ref[0:1, :]
    dest = et_ref[1:2, :]
    nbr = et_ref[2:2 + K, :]
    w1a = w1_ref[:, 0:H]
    w1b = w1_ref[:, H:2 * H]
    w1c = w1_ref[:, 2 * H:3 * H]
    u = (lax.dot_general(curr, w1a, (((1,), (1,)), ((), ())), precision=_PREC)
         + lax.dot_general(dest, w1b, (((1,), (1,)), ((), ())), precision=_PREC)
         + b1_ref[...])
    hh = jnp.maximum(
        lax.dot_general(nbr, w1c, (((1,), (1,)), ((), ())), precision=_PREC)
        + u, 0.0)
    q = jnp.sum(hh * w2_ref[...], axis=1, keepdims=True) + b2_ref[0, 0]
    o_ref[...] = jnp.broadcast_to(q, (K, H))


def _final(embs_t, w1, b1, w2, b2):
    out = pl.pallas_call(
        _final_tile,
        in_specs=[
            pl.BlockSpec((80, F), lambda: (0, 0)),
            pl.BlockSpec((H, 3 * H), lambda: (0, 0)),
            pl.BlockSpec((1, H), lambda: (0, 0)),
            pl.BlockSpec((1, H), lambda: (0, 0)),
            pl.BlockSpec((1, 1), lambda: (0, 0)),
        ],
        out_specs=pl.BlockSpec((K, H), lambda: (0, 0)),
        out_shape=jax.ShapeDtypeStruct((K, H), jnp.float32),
    )(embs_t, w1, b1.reshape(1, H), w2, b2.reshape(1, 1))
    return out[:, :1]


def kernel(x, edge_index, curr_idx, dest_idx, neighbor_indices,
           Wl1, bl1, Wr1, Wl2, bl2, Wr2, W1, b1, W2, b2):
    srcR = edge_index[0].reshape(NS, NCH, CH)
    dstR = edge_index[1].reshape(NS, NCH, CH)
    zrows = jnp.zeros((NPAD, FH), jnp.float32)

    xa = x[:, :FH]
    xb = x[:, FH:]
    aggp1, degp = _seg(xa, xb, srcR, dstR, zrows)
    degp_t = degp.reshape(NS, NDEG)[:, :N].T

    h1s = _dense(aggp1, degp_t, x, Wl1, bl1, Wr1,
                 act=True, split_in=False, split_out=True)

    aggp2, _ = _seg(h1s[0], h1s[1], srcR, dstR, zrows)
    embs = _dense(aggp2, degp_t, h1s, Wl2, bl2, Wr2,
                  act=False, split_in=True, split_out=False)

    tidx = jnp.concatenate([
        jnp.asarray(curr_idx, jnp.int32).reshape(1),
        jnp.asarray(dest_idx, jnp.int32).reshape(1),
        neighbor_indices.astype(jnp.int32),
        jnp.zeros((80 - 2 - K,), jnp.int32),
    ])
    embs_t = _gather_t(embs, tidx)
    return _final(embs_t, W1, b1, W2, b2)


# speedup vs baseline: 10.4260x; 2.1641x over previous
import functools

import jax
import jax.numpy as jnp
from jax import lax
from jax.experimental import pallas as pl
from jax.experimental.pallas import tpu as pltpu
from jax.experimental.pallas import tpu_sc as plsc

N = 10000
E = 320000
F = 128
H = 128
K = 64

NC = 2
NS = 16
NW = NC * NS
EPT = E // NW
CH = 80
NCH = EPT // CH
NP = (NCH - 1) // 2
NPAD = 10016
ZR = NPAD // NS
CR = 624
CR_LAST = N - CR * (NS - 1)
NDEG = 10000

_PREC = jax.lax.Precision.HIGHEST


def _seg_body(table, srcR, dstR, zrows, acc_out, deg_out,
              src_buf, dst_buf, rows0, rows1, deg_loc, acc,
              sg0, sg1, ss0, ss1):
    core = lax.axis_index("c")
    sub = lax.axis_index("s")
    wid = core * NS + sub

    pltpu.sync_copy(zrows.at[pl.ds(sub * ZR, ZR)], acc.at[pl.ds(sub * ZR, ZR)])

    zero16 = jnp.zeros((16,), jnp.float32)

    def _zdeg(i, _):
        deg_loc[pl.ds(i * 16, 16)] = zero16
        return 0

    lax.fori_loop(0, NDEG // 16, _zdeg, 0)

    pltpu.sync_copy(srcR.at[wid], src_buf)
    pltpu.sync_copy(dstR.at[wid], dst_buf)

    plsc.subcore_barrier()

    ones16 = jnp.full((16,), 1.0, jnp.float32)

    def _g_start(j, rbuf, sem):
        pltpu.async_copy(table.at[src_buf.at[j]], rbuf, sem)

    def _g_wait(rbuf, sem):
        pltpu.make_async_copy(table.at[src_buf.at[0]], rbuf, sem).wait()

    def _s_start(j, rbuf, sem):
        pltpu.async_copy(rbuf, acc.at[dst_buf.at[j]], sem, add=True)

    def _s_wait(rbuf, sem):
        pltpu.make_async_copy(rbuf, acc.at[dst_buf.at[0]], sem).wait()

    def _deg(j):
        for q in range(CH // 16):
            d = dst_buf[j, pl.ds(q * 16, 16)]
            plsc.addupdate_scatter(deg_loc, [d], ones16)

    _g_start(0, rows0, sg0)

    def _pair(p, _):
        @pl.when(p > 0)
        def _():
            _s_wait(rows1, ss1)

        _g_start(2 * p + 1, rows1, sg1)
        _g_wait(rows0, sg0)
        _s_start(2 * p, rows0, ss0)
        _deg(2 * p)
        _g_wait(rows1, sg1)
        _s_wait(rows0, ss0)

        @pl.when(p < NP - 1)
        def _():
            _g_start(2 * p + 2, rows0, sg0)

        _s_start(2 * p + 1, rows1, ss1)
        _deg(2 * p + 1)
        return 0

    lax.fori_loop(0, NP, _pair, 0)

    _g_start(NCH - 1, rows0, sg0)
    _s_wait(rows1, ss1)
    _g_wait(rows0, sg0)
    _s_start(NCH - 1, rows0, ss0)
    _deg(NCH - 1)
    _s_wait(rows0, ss0)

    plsc.subcore_barrier()

    @pl.when(sub < NS - 1)
    def _():
        pltpu.sync_copy(acc.at[pl.ds(sub * CR, CR)],
                        acc_out.at[core, pl.ds(sub * CR, CR)])

    @pl.when(sub == NS - 1)
    def _():
        pltpu.sync_copy(acc.at[pl.ds((NS - 1) * CR, CR_LAST)],
                        acc_out.at[core, pl.ds((NS - 1) * CR, CR_LAST)])

    pltpu.sync_copy(deg_loc, deg_out.at[pl.ds(wid * NDEG, NDEG)])


_seg = pl.kernel(
    _seg_body,
    out_type=(
        jax.ShapeDtypeStruct((NC, N, F), jnp.float32),
        jax.ShapeDtypeStruct((NW * NDEG,), jnp.float32),
    ),
    mesh=plsc.VectorSubcoreMesh(core_axis_name="c", subcore_axis_name="s"),
    scratch_types=(
        pltpu.VMEM((NCH, CH), jnp.int32),
        pltpu.VMEM((NCH, CH), jnp.int32),
        pltpu.VMEM((CH, F), jnp.float32),
        pltpu.VMEM((CH, F), jnp.float32),
        pltpu.VMEM((NDEG,), jnp.float32),
        pltpu.VMEM_SHARED((NPAD, F), jnp.float32),
        pltpu.SemaphoreType.DMA,
        pltpu.SemaphoreType.DMA,
        pltpu.SemaphoreType.DMA,
        pltpu.SemaphoreType.DMA,
    ),
    compiler_params=pltpu.CompilerParams(needs_layout_passes=False,
                                         use_tc_tiling_on_sc=False),
)


def _gt_body(embs, tidx, out, tidx_v, rows, sem):
    core = lax.axis_index("c")
    sub = lax.axis_index("s")

    @pl.when(jnp.logical_and(core == 0, sub == 0))
    def _():
        pltpu.sync_copy(tidx, tidx_v)
        pltpu.async_copy(embs.at[tidx_v], rows, sem).wait()
        pltpu.sync_copy(rows, out)


_gather_t = pl.kernel(
    _gt_body,
    out_type=jax.ShapeDtypeStruct((80, F), jnp.float32),
    mesh=plsc.VectorSubcoreMesh(core_axis_name="c", subcore_axis_name="s"),
    scratch_types=(
        pltpu.VMEM((80,), jnp.int32),
        pltpu.VMEM((80, F), jnp.float32),
        pltpu.SemaphoreType.DMA,
    ),
    compiler_params=pltpu.CompilerParams(needs_layout_passes=False),
)


def _dense_tile(aggp_ref, degp_ref, x_ref, wl_ref, wr_ref, b_ref, o_ref, *, act):
    deg = jnp.sum(degp_ref[...], axis=1)
    dinv = 1.0 / jnp.clip(deg, 1.0)[:, None]
    mean = (aggp_ref[0] + aggp_ref[1]) * dinv
    y = (lax.dot_general(mean, wl_ref[...], (((1,), (1,)), ((), ())),
                         precision=_PREC)
         + lax.dot_general(x_ref[...], wr_ref[...], (((1,), (1,)), ((), ())),
                           precision=_PREC)
         + b_ref[...])
    if act:
        y = jnp.maximum(y, 0.0)
    o_ref[...] = y


def _dense(aggp, degp_t, x, wl, b, wr, act):
    blk = 1000
    grid = N // blk
    return pl.pallas_call(
        functools.partial(_dense_tile, act=act),
        grid=(grid,),
        in_specs=[
            pl.BlockSpec((NC, blk, F), lambda i: (0, i, 0)),
            pl.BlockSpec((blk, NW), lambda i: (i, 0)),
            pl.BlockSpec((blk, F), lambda i: (i, 0)),
            pl.BlockSpec((H, F), lambda i: (0, 0)),
            pl.BlockSpec((H, F), lambda i: (0, 0)),
            pl.BlockSpec((1, H), lambda i: (0, 0)),
        ],
        out_specs=pl.BlockSpec((blk, H), lambda i: (i, 0)),
        out_shape=jax.ShapeDtypeStruct((N, H), jnp.float32),
    )(aggp, degp_t, x, wl, wr, b.reshape(1, H))


def _final_tile(et_ref, w1_ref, b1_ref, w2_ref, b2_ref, o_ref):
    curr = et_ref[0:1, :]
    dest = et_ref[1:2, :]
    nbr = et_ref[2:2 + K, :]
    w1a = w1_ref[:, 0:H]
    w1b = w1_ref[:, H:2 * H]
    w1c = w1_ref[:, 2 * H:3 * H]
    u = (lax.dot_general(curr, w1a, (((1,), (1,)), ((), ())), precision=_PREC)
         + lax.dot_general(dest, w1b, (((1,), (1,)), ((), ())), precision=_PREC)
         + b1_ref[...])
    hh = jnp.maximum(
        lax.dot_general(nbr, w1c, (((1,), (1,)), ((), ())), precision=_PREC)
        + u, 0.0)
    q = jnp.sum(hh * w2_ref[...], axis=1, keepdims=True) + b2_ref[0, 0]
    o_ref[...] = jnp.broadcast_to(q, (K, H))


def _final(embs_t, w1, b1, w2, b2):
    out = pl.pallas_call(
        _final_tile,
        in_specs=[
            pl.BlockSpec((80, F), lambda: (0, 0)),
            pl.BlockSpec((H, 3 * H), lambda: (0, 0)),
            pl.BlockSpec((1, H), lambda: (0, 0)),
            pl.BlockSpec((1, H), lambda: (0, 0)),
            pl.BlockSpec((1, 1), lambda: (0, 0)),
        ],
        out_specs=pl.BlockSpec((K, H), lambda: (0, 0)),
        out_shape=jax.ShapeDtypeStruct((K, H), jnp.float32),
    )(embs_t, w1, b1.reshape(1, H), w2, b2.reshape(1, 1))
    return out[:, :1]


def kernel(x, edge_index, curr_idx, dest_idx, neighbor_indices,
           Wl1, bl1, Wr1, Wl2, bl2, Wr2, W1, b1, W2, b2):
    srcR = edge_index[0].reshape(NW, NCH, CH)
    dstR = edge_index[1].reshape(NW, NCH, CH)
    zrows = jnp.zeros((NPAD, F), jnp.float32)

    aggp1, degp = _seg(x, srcR, dstR, zrows)
    degp_t = degp.reshape(NW, NDEG)[:, :N].T

    h1 = _dense(aggp1, degp_t, x, Wl1, bl1, Wr1, act=True)

    aggp2, _ = _seg(h1, srcR, dstR, zrows)
    embs = _dense(aggp2, degp_t, h1, Wl2, bl2, Wr2, act=False)

    tidx = jnp.concatenate([
        jnp.asarray(curr_idx, jnp.int32).reshape(1),
        jnp.asarray(dest_idx, jnp.int32).reshape(1),
        neighbor_indices.astype(jnp.int32),
        jnp.zeros((80 - 2 - K,), jnp.int32),
    ])
    embs_t = _gather_t(embs, tidx)
    return _final(embs_t, W1, b1, W2, b2)


# speedup vs baseline: 10.7990x; 1.0358x over previous
import functools

import jax
import jax.numpy as jnp
from jax import lax
from jax.experimental import pallas as pl
from jax.experimental.pallas import tpu as pltpu
from jax.experimental.pallas import tpu_sc as plsc

N = 10000
E = 320000
F = 128
H = 128
K = 64

NC = 2
NS = 16
NW = NC * NS
EPT = E // NW
CH = 80
NCH = EPT // CH
NP = (NCH - 1) // 2
NPAD = 10016
ZR = NPAD // NS
CR = 624
CR_LAST = N - CR * (NS - 1)
NDEG = 10000

_PREC = jax.lax.Precision.HIGHEST


def _seg_body(table, srcR, dstR, zrows, acc_out, deg_out,
              src_buf, dst_buf, rows0, rows1, deg_loc, acc,
              sg0, sg1, ss0, ss1):
    core = lax.axis_index("c")
    sub = lax.axis_index("s")
    wid = core * NS + sub

    pltpu.sync_copy(zrows.at[pl.ds(sub * ZR, ZR)], acc.at[pl.ds(sub * ZR, ZR)])

    zero16 = jnp.zeros((16,), jnp.float32)

    def _zdeg(i, _):
        deg_loc[pl.ds(i * 16, 16)] = zero16
        return 0

    lax.fori_loop(0, NDEG // 16, _zdeg, 0)

    pltpu.sync_copy(srcR.at[wid], src_buf)
    pltpu.sync_copy(dstR.at[wid], dst_buf)

    plsc.subcore_barrier()

    ones16 = jnp.full((16,), 1.0, jnp.float32)

    def _g_start(j, rbuf, sem):
        pltpu.async_copy(table.at[src_buf.at[j]], rbuf, sem)

    def _g_wait(rbuf, sem):
        pltpu.make_async_copy(table.at[src_buf.at[0]], rbuf, sem).wait()

    def _s_start(j, rbuf, sem):
        pltpu.async_copy(rbuf, acc.at[dst_buf.at[j]], sem, add=True)

    def _s_wait(rbuf, sem):
        pltpu.make_async_copy(rbuf, acc.at[dst_buf.at[0]], sem).wait()

    def _deg(j):
        for q in range(CH // 16):
            d = dst_buf[j, pl.ds(q * 16, 16)]
            plsc.addupdate_scatter(deg_loc, [d], ones16)

    _g_start(0, rows0, sg0)

    def _pair(p, _):
        @pl.when(p > 0)
        def _():
            _s_wait(rows1, ss1)

        _g_start(2 * p + 1, rows1, sg1)
        _g_wait(rows0, sg0)
        _s_start(2 * p, rows0, ss0)
        _deg(2 * p)
        _g_wait(rows1, sg1)
        _s_wait(rows0, ss0)

        @pl.when(p < NP - 1)
        def _():
            _g_start(2 * p + 2, rows0, sg0)

        _s_start(2 * p + 1, rows1, ss1)
        _deg(2 * p + 1)
        return 0

    lax.fori_loop(0, NP, _pair, 0)

    _g_start(NCH - 1, rows0, sg0)
    _s_wait(rows1, ss1)
    _g_wait(rows0, sg0)
    _s_start(NCH - 1, rows0, ss0)
    _deg(NCH - 1)
    _s_wait(rows0, ss0)

    plsc.subcore_barrier()

    @pl.when(sub < NS - 1)
    def _():
        pltpu.sync_copy(acc.at[pl.ds(sub * CR, CR)],
                        acc_out.at[core, pl.ds(sub * CR, CR)])

    @pl.when(sub == NS - 1)
    def _():
        pltpu.sync_copy(acc.at[pl.ds((NS - 1) * CR, CR_LAST)],
                        acc_out.at[core, pl.ds((NS - 1) * CR, CR_LAST)])

    pltpu.sync_copy(deg_loc, deg_out.at[pl.ds(wid * NDEG, NDEG)])


_seg = pl.kernel(
    _seg_body,
    out_type=(
        jax.ShapeDtypeStruct((NC, N, F), jnp.float32),
        jax.ShapeDtypeStruct((NW * NDEG,), jnp.float32),
    ),
    mesh=plsc.VectorSubcoreMesh(core_axis_name="c", subcore_axis_name="s"),
    scratch_types=(
        pltpu.VMEM((NCH, CH), jnp.int32),
        pltpu.VMEM((NCH, CH), jnp.int32),
        pltpu.VMEM((CH, F), jnp.float32),
        pltpu.VMEM((CH, F), jnp.float32),
        pltpu.VMEM((NDEG,), jnp.float32),
        pltpu.VMEM_SHARED((NPAD, F), jnp.float32),
        pltpu.SemaphoreType.DMA,
        pltpu.SemaphoreType.DMA,
        pltpu.SemaphoreType.DMA,
        pltpu.SemaphoreType.DMA,
    ),
    compiler_params=pltpu.CompilerParams(needs_layout_passes=False,
                                         use_tc_tiling_on_sc=False),
)


def _gt_body(aggp2, h1, degp_t, tidx, a_out, h_out, d_out,
             tidx_v, arows, hrows, drows, sem):
    core = lax.axis_index("c")
    sub = lax.axis_index("s")

    @pl.when(jnp.logical_and(core == 0, sub == 0))
    def _():
        pltpu.sync_copy(tidx, tidx_v)
        pltpu.async_copy(aggp2.at[0].at[tidx_v], arows, sem).wait()
        pltpu.sync_copy(arows, a_out.at[0])
        pltpu.async_copy(aggp2.at[1].at[tidx_v], arows, sem).wait()
        pltpu.sync_copy(arows, a_out.at[1])
        pltpu.async_copy(h1.at[tidx_v], hrows, sem).wait()
        pltpu.sync_copy(hrows, h_out)
        pltpu.async_copy(degp_t.at[tidx_v], drows, sem).wait()
        pltpu.sync_copy(drows, d_out)


_gather_t = pl.kernel(
    _gt_body,
    out_type=(
        jax.ShapeDtypeStruct((NC, 80, F), jnp.float32),
        jax.ShapeDtypeStruct((80, F), jnp.float32),
        jax.ShapeDtypeStruct((80, NW), jnp.float32),
    ),
    mesh=plsc.VectorSubcoreMesh(core_axis_name="c", subcore_axis_name="s"),
    scratch_types=(
        pltpu.VMEM((80,), jnp.int32),
        pltpu.VMEM((80, F), jnp.float32),
        pltpu.VMEM((80, F), jnp.float32),
        pltpu.VMEM((80, NW), jnp.float32),
        pltpu.SemaphoreType.DMA,
    ),
    compiler_params=pltpu.CompilerParams(needs_layout_passes=False,
                                         use_tc_tiling_on_sc=False),
)


def _dense_tile(aggp_ref, degp_ref, x_ref, wl_ref, wr_ref, b_ref, o_ref, *, act):
    deg = jnp.sum(degp_ref[...], axis=1)
    dinv = 1.0 / jnp.clip(deg, 1.0)[:, None]
    mean = (aggp_ref[0] + aggp_ref[1]) * dinv
    y = (lax.dot_general(mean, wl_ref[...], (((1,), (1,)), ((), ())),
                         precision=_PREC)
         + lax.dot_general(x_ref[...], wr_ref[...], (((1,), (1,)), ((), ())),
                           precision=_PREC)
         + b_ref[...])
    if act:
        y = jnp.maximum(y, 0.0)
    o_ref[...] = y


def _dense(aggp, degp_t, x, wl, b, wr, act):
    blk = 1000
    grid = N // blk
    return pl.pallas_call(
        functools.partial(_dense_tile, act=act),
        grid=(grid,),
        in_specs=[
            pl.BlockSpec((NC, blk, F), lambda i: (0, i, 0)),
            pl.BlockSpec((blk, NW), lambda i: (i, 0)),
            pl.BlockSpec((blk, F), lambda i: (i, 0)),
            pl.BlockSpec((H, F), lambda i: (0, 0)),
            pl.BlockSpec((H, F), lambda i: (0, 0)),
            pl.BlockSpec((1, H), lambda i: (0, 0)),
        ],
        out_specs=pl.BlockSpec((blk, H), lambda i: (i, 0)),
        out_shape=jax.ShapeDtypeStruct((N, H), jnp.float32),
    )(aggp, degp_t, x, wl, wr, b.reshape(1, H))


def _final_tile(a_ref, h_ref, d_ref, wl_ref, bl_ref, wr_ref,
                w1_ref, b1_ref, w2_ref, b2_ref, o_ref):
    deg = jnp.sum(d_ref[...], axis=1)
    dinv = 1.0 / jnp.clip(deg, 1.0)[:, None]
    mean = (a_ref[0] + a_ref[1]) * dinv
    emb = (lax.dot_general(mean, wl_ref[...], (((1,), (1,)), ((), ())),
                           precision=_PREC)
           + lax.dot_general(h_ref[...], wr_ref[...], (((1,), (1,)), ((), ())),
                             precision=_PREC)
           + bl_ref[...])
    curr = emb[0:1, :]
    dest = emb[1:2, :]
    nbr = emb[2:2 + K, :]
    w1a = w1_ref[:, 0:H]
    w1b = w1_ref[:, H:2 * H]
    w1c = w1_ref[:, 2 * H:3 * H]
    u = (lax.dot_general(curr, w1a, (((1,), (1,)), ((), ())), precision=_PREC)
         + lax.dot_general(dest, w1b, (((1,), (1,)), ((), ())), precision=_PREC)
         + b1_ref[...])
    hh = jnp.maximum(
        lax.dot_general(nbr, w1c, (((1,), (1,)), ((), ())), precision=_PREC)
        + u, 0.0)
    q = jnp.sum(hh * w2_ref[...], axis=1, keepdims=True) + b2_ref[0, 0]
    o_ref[...] = jnp.broadcast_to(q, (K, H))


def _final(a_t, h_t, d_t, wl, bl, wr, w1, b1, w2, b2):
    out = pl.pallas_call(
        _final_tile,
        in_specs=[
            pl.BlockSpec((NC, 80, F), lambda: (0, 0, 0)),
            pl.BlockSpec((80, F), lambda: (0, 0)),
            pl.BlockSpec((80, NW), lambda: (0, 0)),
            pl.BlockSpec((H, F), lambda: (0, 0)),
            pl.BlockSpec((1, H), lambda: (0, 0)),
            pl.BlockSpec((H, F), lambda: (0, 0)),
            pl.BlockSpec((H, 3 * H), lambda: (0, 0)),
            pl.BlockSpec((1, H), lambda: (0, 0)),
            pl.BlockSpec((1, H), lambda: (0, 0)),
            pl.BlockSpec((1, 1), lambda: (0, 0)),
        ],
        out_specs=pl.BlockSpec((K, H), lambda: (0, 0)),
        out_shape=jax.ShapeDtypeStruct((K, H), jnp.float32),
    )(a_t, h_t, d_t, wl, bl.reshape(1, H), wr,
      w1, b1.reshape(1, H), w2, b2.reshape(1, 1))
    return out[:, :1]


def kernel(x, edge_index, curr_idx, dest_idx, neighbor_indices,
           Wl1, bl1, Wr1, Wl2, bl2, Wr2, W1, b1, W2, b2):
    srcR = edge_index[0].reshape(NW, NCH, CH)
    dstR = edge_index[1].reshape(NW, NCH, CH)
    zrows = jnp.zeros((NPAD, F), jnp.float32)

    aggp1, degp = _seg(x, srcR, dstR, zrows)
    degp_t = degp.reshape(NW, NDEG).T

    h1 = _dense(aggp1, degp_t, x, Wl1, bl1, Wr1, act=True)

    aggp2, _ = _seg(h1, srcR, dstR, zrows)

    tidx = jnp.concatenate([
        jnp.asarray(curr_idx, jnp.int32).reshape(1),
        jnp.asarray(dest_idx, jnp.int32).reshape(1),
        neighbor_indices.astype(jnp.int32),
        jnp.zeros((80 - 2 - K,), jnp.int32),
    ])
    a_t, h_t, d_t = _gather_t(aggp2, h1, degp_t, tidx)
    return _final(a_t, h_t, d_t, Wl2, bl2, Wr2, W1, b1, W2, b2)


# speedup vs baseline: 10.9566x; 1.0146x over previous
import functools

import jax
import jax.numpy as jnp
from jax import lax
from jax.experimental import pallas as pl
from jax.experimental.pallas import tpu as pltpu
from jax.experimental.pallas import tpu_sc as plsc

N = 10000
E = 320000
F = 128
H = 128
K = 64

NC = 2
NS = 16
NW = NC * NS
EPT = E // NW
CH = 80
NCH = EPT // CH
SELCH = 128
NSEL = EPT // SELCH + 1
NPAD = 10016
ZR = NPAD // NS
CR = 624
CR_LAST = N - CR * (NS - 1)
PKS = 14

_PREC = jax.lax.Precision.HIGHEST
_SC_PARAMS = pltpu.CompilerParams(needs_layout_passes=False,
                                  use_tc_tiling_on_sc=False)


def _flags_body(srcR, dstR, tidx, zi32, zf32, epk_out, flags_out, deg_out,
                dT_out, src_buf, dst_buf, epk_loc, tidx_v, flag_t, flag_s,
                deg_loc, dvec, mbufd, dsums):
    core = lax.axis_index("c")
    sub = lax.axis_index("s")
    wid = core * NS + sub

    pltpu.sync_copy(zi32, flag_t)
    pltpu.sync_copy(zi32, flag_s)
    pltpu.sync_copy(zf32, deg_loc)
    pltpu.sync_copy(tidx, tidx_v)
    pltpu.sync_copy(srcR.at[wid], src_buf)
    pltpu.sync_copy(dstR.at[wid], dst_buf)

    ones16i = jnp.full((16,), 1, jnp.int32)
    ones16f = jnp.full((16,), 1.0, jnp.float32)
    for t in range(5):
        tv = tidx_v[pl.ds(t * 16, 16)]
        plsc.store_scatter(flag_t, [tv], ones16i)
        plsc.store_scatter(flag_s, [tv], ones16i)

    def outer(j, _):
        for q in range(CH // 16):
            d = dst_buf[j, pl.ds(q * 16, 16)]
            s = src_buf[j, pl.ds(q * 16, 16)]
            epk_loc[j, pl.ds(q * 16, 16)] = jnp.left_shift(s, PKS) | d
            plsc.addupdate_scatter(deg_loc, [d], ones16f)
            ft = plsc.load_gather(flag_t, [d])
            plsc.store_scatter(flag_s, [s], ones16i, mask=ft > 0)
        return 0

    lax.fori_loop(0, NCH, outer, 0)

    for t in range(5):
        tv = tidx_v[pl.ds(t * 16, 16)]
        dvec[pl.ds(t * 16, 16)] = plsc.load_gather(deg_loc, [tv])
    pltpu.sync_copy(dvec, dsums.at[sub])
    plsc.subcore_barrier()

    @pl.when(sub == 0)
    def _():
        pltpu.sync_copy(dsums, mbufd)
        for t in range(5):
            v = mbufd[0, pl.ds(t * 16, 16)]
            for kk in range(1, NS):
                v = v + mbufd[kk, pl.ds(t * 16, 16)]
            dvec[pl.ds(t * 16, 16)] = v
        pltpu.sync_copy(dvec, dT_out.at[core])

    pltpu.sync_copy(epk_loc, epk_out.at[wid])
    pltpu.sync_copy(flag_s, flags_out.at[wid])
    pltpu.sync_copy(deg_loc, deg_out.at[pl.ds(wid * N, N)])


_flags = pl.kernel(
    _flags_body,
    out_type=(
        jax.ShapeDtypeStruct((NW, NCH, CH), jnp.int32),
        jax.ShapeDtypeStruct((NW, N), jnp.int32),
        jax.ShapeDtypeStruct((NW * N,), jnp.float32),
        jax.ShapeDtypeStruct((NC, 80), jnp.float32),
    ),
    mesh=plsc.VectorSubcoreMesh(core_axis_name="c", subcore_axis_name="s"),
    scratch_types=(
        pltpu.VMEM((NCH, CH), jnp.int32),
        pltpu.VMEM((NCH, CH), jnp.int32),
        pltpu.VMEM((NCH, CH), jnp.int32),
        pltpu.VMEM((80,), jnp.int32),
        pltpu.VMEM((N,), jnp.int32),
        pltpu.VMEM((N,), jnp.int32),
        pltpu.VMEM((N,), jnp.float32),
        pltpu.VMEM((80,), jnp.float32),
        pltpu.VMEM((NS, 80), jnp.float32),
        pltpu.VMEM_SHARED((NS, 80), jnp.float32),
    ),
    compiler_params=_SC_PARAMS,
)


def _or_tile(f_ref, o_ref):
    v = f_ref[0]
    for kk in range(1, NW):
        v = v | f_ref[kk]
    o_ref[0] = v


def _or_merge(flags):
    return pl.pallas_call(
        _or_tile,
        in_specs=[pl.BlockSpec((NW, N), lambda: (0, 0))],
        out_specs=pl.BlockSpec((1, N), lambda: (0, 0)),
        out_shape=jax.ShapeDtypeStruct((1, N), jnp.int32),
    )(flags)


def _seg1_body(table, epkR, zrows, flag_m, acc_out,
               epk_buf, fS, selpk, gsrc, gdst, rows, acc, sem):
    core = lax.axis_index("c")
    sub = lax.axis_index("s")
    wid = core * NS + sub

    pltpu.sync_copy(zrows.at[pl.ds(sub * ZR, ZR)], acc.at[pl.ds(sub * ZR, ZR)])
    pltpu.sync_copy(epkR.at[wid], epk_buf)
    pltpu.sync_copy(flag_m.at[0], fS)

    dmask = jnp.full((16,), (1 << PKS) - 1, jnp.int32)

    def outer(j, cnt):
        for q in range(CH // 16):
            w = epk_buf[j, pl.ds(q * 16, 16)]
            d = w & dmask
            m = plsc.load_gather(fS, [d]) > 0
            mi = jnp.where(m, 1, 0).astype(jnp.int32)
            c = plsc.cumsum(mi)
            pos = cnt + c - 1
            row = jnp.right_shift(pos, 7)
            col = jnp.bitwise_and(pos, SELCH - 1)
            plsc.store_scatter(selpk, [row, col], w, mask=m)
            cnt = cnt + plsc.all_reduce_population_count(m)
        return cnt

    cnt = lax.fori_loop(0, NCH, outer, jnp.zeros((16,), jnp.int32))
    cs = jnp.max(cnt)

    nch = jnp.right_shift(cs + SELCH - 1, 7)
    total = nch * SELCH
    iota = lax.iota(jnp.int32, 16)
    pad = jnp.full((16,), N, jnp.int32)
    for t in range(8):
        lp = cs + t * 16 + iota
        m2 = lp < total
        row = jnp.right_shift(lp, 7)
        col = jnp.bitwise_and(lp, SELCH - 1)
        plsc.store_scatter(selpk, [row, col], pad, mask=m2)

    plsc.subcore_barrier()

    def _ch(j, _):
        for q in range(SELCH // 16):
            w = selpk[j, pl.ds(q * 16, 16)]
            gsrc[pl.ds(q * 16, 16)] = jnp.right_shift(w, PKS)
            gdst[pl.ds(q * 16, 16)] = w & dmask
        pltpu.async_copy(table.at[gsrc], rows, sem).wait()
        pltpu.sync_copy(rows, acc.at[gdst], add=True)
        return 0

    lax.fori_loop(0, nch, _ch, 0)

    plsc.subcore_barrier()

    @pl.when(sub < NS - 1)
    def _():
        pltpu.sync_copy(acc.at[pl.ds(sub * CR, CR)],
                        acc_out.at[core, pl.ds(sub * CR, CR)])

    @pl.when(sub == NS - 1)
    def _():
        pltpu.sync_copy(acc.at[pl.ds((NS - 1) * CR, CR_LAST)],
                        acc_out.at[core, pl.ds((NS - 1) * CR, CR_LAST)])


_seg1 = pl.kernel(
    _seg1_body,
    out_type=jax.ShapeDtypeStruct((NC, N, F), jnp.float32),
    mesh=plsc.VectorSubcoreMesh(core_axis_name="c", subcore_axis_name="s"),
    scratch_types=(
        pltpu.VMEM((NCH, CH), jnp.int32),
        pltpu.VMEM((N,), jnp.int32),
        pltpu.VMEM((NSEL, SELCH), jnp.int32),
        pltpu.VMEM((SELCH,), jnp.int32),
        pltpu.VMEM((SELCH,), jnp.int32),
        pltpu.VMEM((SELCH, F), jnp.float32),
        pltpu.VMEM_SHARED((NPAD, F), jnp.float32),
        pltpu.SemaphoreType.DMA,
    ),
    compiler_params=_SC_PARAMS,
)


def _seg2_body(h1, epkR, tidx, zrows, a_out, h_out,
               epk_buf, tidx_v, flag_t, selpk, gsrc, gdst, rows,
               slot_buf, arows, hrows, acc, sem):
    core = lax.axis_index("c")
    sub = lax.axis_index("s")
    wid = core * NS + sub

    pltpu.sync_copy(tidx, tidx_v)
    pltpu.sync_copy(epkR.at[wid], epk_buf)

    zero16i = jnp.zeros((16,), jnp.int32)

    def _zf(i, _):
        flag_t[pl.ds(i * 16, 16)] = zero16i
        return 0

    lax.fori_loop(0, N // 16, _zf, 0)
    iota = lax.iota(jnp.int32, 16)
    for t in range(5):
        tv = tidx_v[pl.ds(t * 16, 16)]
        plsc.store_scatter(flag_t, [tv], iota + (t * 16 + 1))

    @pl.when(sub == 0)
    def _():
        pltpu.sync_copy(zrows.at[pl.ds(0, SELCH)], acc)

    @pl.when(jnp.logical_and(core == 0, sub == 1))
    def _():
        pltpu.async_copy(h1.at[tidx_v], hrows, sem).wait()
        pltpu.sync_copy(hrows, h_out)

    dmask = jnp.full((16,), (1 << PKS) - 1, jnp.int32)

    def outer(j, cnt):
        for q in range(CH // 16):
            w = epk_buf[j, pl.ds(q * 16, 16)]
            d = w & dmask
            g = plsc.load_gather(flag_t, [d])
            m = g > 0
            mi = jnp.where(m, 1, 0).astype(jnp.int32)
            c = plsc.cumsum(mi)
            pos = cnt + c - 1
            row = jnp.right_shift(pos, 7)
            col = jnp.bitwise_and(pos, SELCH - 1)
            wslot = (w & ~dmask) | (g - 1)
            plsc.store_scatter(selpk, [row, col], wslot, mask=m)
            cnt = cnt + plsc.all_reduce_population_count(m)
        return cnt

    cnt = lax.fori_loop(0, NCH, outer, jnp.zeros((16,), jnp.int32))
    cs = jnp.max(cnt)
    nch = jnp.right_shift(cs + SELCH - 1, 7)
    total = nch * SELCH
    pad = jnp.full((16,), 96, jnp.int32)
    for t in range(8):
        lp = cs + t * 16 + iota
        m2 = lp < total
        row = jnp.right_shift(lp, 7)
        col = jnp.bitwise_and(lp, SELCH - 1)
        plsc.store_scatter(selpk, [row, col], pad, mask=m2)

    plsc.subcore_barrier()

    def _ch(j, _):
        for q in range(SELCH // 16):
            w = selpk[j, pl.ds(q * 16, 16)]
            gsrc[pl.ds(q * 16, 16)] = jnp.right_shift(w, PKS)
            gdst[pl.ds(q * 16, 16)] = w & dmask
        pltpu.async_copy(h1.at[gsrc], rows, sem).wait()
        pltpu.sync_copy(rows, acc.at[gdst], add=True)
        return 0

    lax.fori_loop(0, nch, _ch, 0)

    plsc.subcore_barrier()

    @pl.when(sub == 0)
    def _():
        for t in range(5):
            tv = tidx_v[pl.ds(t * 16, 16)]
            slot_buf[pl.ds(t * 16, 16)] = plsc.load_gather(flag_t, [tv]) - 1
        pltpu.async_copy(acc.at[slot_buf], arows, sem).wait()
        pltpu.sync_copy(arows, a_out.at[core])


_seg2 = pl.kernel(
    _seg2_body,
    out_type=(
        jax.ShapeDtypeStruct((NC, 80, F), jnp.float32),
        jax.ShapeDtypeStruct((80, F), jnp.float32),
    ),
    mesh=plsc.VectorSubcoreMesh(core_axis_name="c", subcore_axis_name="s"),
    scratch_types=(
        pltpu.VMEM((NCH, CH), jnp.int32),
        pltpu.VMEM((80,), jnp.int32),
        pltpu.VMEM((N,), jnp.int32),
        pltpu.VMEM((NSEL, SELCH), jnp.int32),
        pltpu.VMEM((SELCH,), jnp.int32),
        pltpu.VMEM((SELCH,), jnp.int32),
        pltpu.VMEM((SELCH, F), jnp.float32),
        pltpu.VMEM((80,), jnp.int32),
        pltpu.VMEM((80, F), jnp.float32),
        pltpu.VMEM((80, F), jnp.float32),
        pltpu.VMEM_SHARED((SELCH, F), jnp.float32),
        pltpu.SemaphoreType.DMA,
    ),
    compiler_params=_SC_PARAMS,
)


def _dense_tile(aggp_ref, degp_ref, x_ref, wl_ref, wr_ref, b_ref, o_ref, *, act):
    deg = jnp.sum(degp_ref[...], axis=1)
    dinv = 1.0 / jnp.clip(deg, 1.0)[:, None]
    mean = (aggp_ref[0] + aggp_ref[1]) * dinv
    y = (lax.dot_general(mean, wl_ref[...], (((1,), (1,)), ((), ())),
                         precision=_PREC)
         + lax.dot_general(x_ref[...], wr_ref[...], (((1,), (1,)), ((), ())),
                           precision=_PREC)
         + b_ref[...])
    if act:
        y = jnp.maximum(y, 0.0)
    o_ref[...] = y


def _dense(aggp, degp_t, x, wl, b, wr, act):
    blk = 1000
    grid = N // blk
    return pl.pallas_call(
        functools.partial(_dense_tile, act=act),
        grid=(grid,),
        in_specs=[
            pl.BlockSpec((NC, blk, F), lambda i: (0, i, 0)),
            pl.BlockSpec((blk, NW), lambda i: (i, 0)),
            pl.BlockSpec((blk, F), lambda i: (i, 0)),
            pl.BlockSpec((H, F), lambda i: (0, 0)),
            pl.BlockSpec((H, F), lambda i: (0, 0)),
            pl.BlockSpec((1, H), lambda i: (0, 0)),
        ],
        out_specs=pl.BlockSpec((blk, H), lambda i: (i, 0)),
        out_shape=jax.ShapeDtypeStruct((N, H), jnp.float32),
    )(aggp, degp_t, x, wl, wr, b.reshape(1, H))


def _final_tile(a_ref, h_ref, d_ref, wl_ref, bl_ref, wr_ref,
                w1_ref, b1_ref, w2_ref, b2_ref, o_ref):
    deg = d_ref[0] + d_ref[1]
    dinv = 1.0 / jnp.clip(deg, 1.0)[:, None]
    mean = (a_ref[0] + a_ref[1]) * dinv
    emb = (lax.dot_general(mean, wl_ref[...], (((1,), (1,)), ((), ())),
                           precision=_PREC)
           + lax.dot_general(h_ref[...], wr_ref[...], (((1,), (1,)), ((), ())),
                             precision=_PREC)
           + bl_ref[...])
    curr = emb[0:1, :]
    dest = emb[1:2, :]
    nbr = emb[2:2 + K, :]
    w1a = w1_ref[:, 0:H]
    w1b = w1_ref[:, H:2 * H]
    w1c = w1_ref[:, 2 * H:3 * H]
    u = (lax.dot_general(curr, w1a, (((1,), (1,)), ((), ())), precision=_PREC)
         + lax.dot_general(dest, w1b, (((1,), (1,)), ((), ())), precision=_PREC)
         + b1_ref[...])
    hh = jnp.maximum(
        lax.dot_general(nbr, w1c, (((1,), (1,)), ((), ())), precision=_PREC)
        + u, 0.0)
    q = jnp.sum(hh * w2_ref[...], axis=1, keepdims=True) + b2_ref[0, 0]
    o_ref[...] = jnp.broadcast_to(q, (K, H))


def _final(a_t, h_t, d_t, wl, bl, wr, w1, b1, w2, b2):
    out = pl.pallas_call(
        _final_tile,
        in_specs=[
            pl.BlockSpec((NC, 80, F), lambda: (0, 0, 0)),
            pl.BlockSpec((80, F), lambda: (0, 0)),
            pl.BlockSpec((NC, 80), lambda: (0, 0)),
            pl.BlockSpec((H, F), lambda: (0, 0)),
            pl.BlockSpec((1, H), lambda: (0, 0)),
            pl.BlockSpec((H, F), lambda: (0, 0)),
            pl.BlockSpec((H, 3 * H), lambda: (0, 0)),
            pl.BlockSpec((1, H), lambda: (0, 0)),
            pl.BlockSpec((1, H), lambda: (0, 0)),
            pl.BlockSpec((1, 1), lambda: (0, 0)),
        ],
        out_specs=pl.BlockSpec((K, H), lambda: (0, 0)),
        out_shape=jax.ShapeDtypeStruct((K, H), jnp.float32),
    )(a_t, h_t, d_t, wl, bl.reshape(1, H), wr,
      w1, b1.reshape(1, H), w2, b2.reshape(1, 1))
    return out[:, :1]


def kernel(x, edge_index, curr_idx, dest_idx, neighbor_indices,
           Wl1, bl1, Wr1, Wl2, bl2, Wr2, W1, b1, W2, b2):
    srcR = edge_index[0].reshape(NW, NCH, CH)
    dstR = edge_index[1].reshape(NW, NCH, CH)
    zrows = jnp.zeros((NPAD, F), jnp.float32)
    zi32 = jnp.zeros((N,), jnp.int32)
    zf32 = jnp.zeros((N,), jnp.float32)

    tidx = jnp.concatenate([
        jnp.asarray(curr_idx, jnp.int32).reshape(1),
        jnp.asarray(dest_idx, jnp.int32).reshape(1),
        neighbor_indices.astype(jnp.int32),
        jnp.zeros((80 - 2 - K,), jnp.int32),
    ])

    epkR, flags, degp, d_t = _flags(srcR, dstR, tidx, zi32, zf32)
    flag_m = _or_merge(flags)
    degp_t = degp.reshape(NW, N).T

    aggp1 = _seg1(x, epkR, zrows, flag_m)
    h1 = _dense(aggp1, degp_t, x, Wl1, bl1, Wr1, act=True)

    a_t, h_t = _seg2(h1, epkR, tidx, zrows)
    return _final(a_t, h_t, d_t, Wl2, bl2, Wr2, W1, b1, W2, b2)
